# Initial kernel scaffold; baseline (speedup 1.0000x reference)
import jax
import jax.numpy as jnp
from jax.experimental import pallas as pl


def kernel(z_batch, codebook, iter):
    raise NotImplementedError("write your pallas kernel here")


# speedup vs baseline: 1.0357x; 1.0357x over previous
import jax
import jax.numpy as jnp
from jax import lax
from jax.experimental import pallas as pl

K = 8192
L = 128
B = 64
LOSS_SCALE = 1.25 / (B * L)


def _vq_body(z_ref, c_ref, zq_ref, idx_ref, loss_ref):
    l = pl.program_id(0)
    z = z_ref[0, 0, :]
    c = c_ref[0, 0, :]
    s = c[None, :] - z[:, None]
    d = jnp.abs(s)
    m = jnp.min(d, axis=1, keepdims=True)
    kio = lax.broadcasted_iota(jnp.int32, (B, K), 1)
    cand = jnp.where(d == m, kio, K)
    idx = jnp.min(cand, axis=1, keepdims=True)
    win = cand == idx
    sw = jnp.sum(jnp.where(win, s, 0.0), axis=1)
    zq_ref[0, 0, :] = z + sw
    idx_ref[0, 0, :] = idx[:, 0]

    part = (jnp.sum(sw * sw) * LOSS_SCALE).reshape(1, 1)

    @pl.when(l == 0)
    def _():
        loss_ref[:, :] = jnp.zeros((1, 1), jnp.float32)

    loss_ref[:, :] += part


def kernel(z_batch, codebook, iter):
    zT = z_batch.T.reshape(L, 1, B)
    cb = codebook.reshape(L, 1, K)
    zq, idx, loss = pl.pallas_call(
        _vq_body,
        grid=(L,),
        in_specs=[
            pl.BlockSpec((1, 1, B), lambda l: (l, 0, 0)),
            pl.BlockSpec((1, 1, K), lambda l: (l, 0, 0)),
        ],
        out_specs=[
            pl.BlockSpec((1, 1, B), lambda l: (l, 0, 0)),
            pl.BlockSpec((1, 1, B), lambda l: (l, 0, 0)),
            pl.BlockSpec((1, 1), lambda l: (0, 0)),
        ],
        out_shape=[
            jax.ShapeDtypeStruct((L, 1, B), jnp.float32),
            jax.ShapeDtypeStruct((L, 1, B), jnp.int32),
            jax.ShapeDtypeStruct((1, 1), jnp.float32),
        ],
    )(zT, cb)
    z_q_st = zq.reshape(L, B).T
    min_idx = idx.reshape(L, B).T
    return (z_q_st, loss[0, 0], min_idx)
